# baseline (speedup 1.0000x reference)
import functools

import numpy as np

import jax
import jax.numpy as jnp
from jax import lax
from jax.experimental import pallas as pl
from jax.experimental.pallas import tpu as pltpu
from jax.experimental.pallas import tpu_sc as plsc

NC = 2
NS = 16
NW = NC * NS
L = 16
G = 64
BLK = 1024


@functools.lru_cache(maxsize=1)
def _mesh():
    return plsc.VectorSubcoreMesh(core_axis_name="c", subcore_axis_name="s",
                                  num_cores=NC, num_subcores=NS)

_SC_PARAMS = pltpu.CompilerParams(needs_layout_passes=False)


def _deg_body(E, Np, dst_hbm, out_hbm, dstbuf, bins):
    epw = E // NW
    c = lax.axis_index("c")
    s = lax.axis_index("s")
    w = s * NC + c
    pltpu.sync_copy(dst_hbm.at[pl.ds(w * epw, epw)], dstbuf.at[pl.ds(0, epw)])

    def zero(i, _):
        bins[pl.ds(i * L, L)] = jnp.zeros((L,), jnp.float32)
        return 0
    lax.fori_loop(0, Np // L, zero, 0)

    ones = jnp.ones((L,), jnp.float32)

    def upd(i, _):
        idx = dstbuf[pl.ds(i * L, L)]
        plsc.addupdate_scatter(bins, [idx], ones)
        return 0
    nfull = epw // L
    lax.fori_loop(0, nfull, upd, 0)
    rem = epw - nfull * L
    if rem:
        idx = dstbuf[pl.ds(nfull * L, L)]
        mask = lax.iota(jnp.int32, L) < rem
        plsc.addupdate_scatter(bins, [idx], ones, mask=mask)
    pltpu.sync_copy(bins, out_hbm.at[w])


def _degree_partials(dst, Np):
    E = dst.shape[0]
    epw = E // NW
    pad = (-epw) % L
    fn = pl.kernel(
        functools.partial(_deg_body, E, Np),
        out_type=jax.ShapeDtypeStruct((NW, Np), jnp.float32),
        mesh=_mesh(),
        compiler_params=_SC_PARAMS,
        scratch_types=[
            pltpu.VMEM((epw + pad,), jnp.int32),
            pltpu.VMEM((Np,), jnp.float32),
        ],
    )
    return fn(dst)


def _agg_body(nstage, rpt, ch, y0, y1, srcr, dstr, zeros_h, s0, s1,
              srcbuf, dstbuf, rows0, rows1, rows2,
              g0, g1, g2, t0, t1, t2, acc):
    c = lax.axis_index("c")
    s = lax.axis_index("s")
    hf = srcbuf.shape[0]
    rows = (rows0, rows1, rows2)
    gsem = (g0, g1, g2)
    ssem = (t0, t1, t2)

    def gather(j, p):
        @pl.when(c == 0)
        def _():
            pltpu.async_copy(y0.at[srcbuf.at[j]], rows[p], gsem[p])

        @pl.when(c == 1)
        def _():
            pltpu.async_copy(y1.at[srcbuf.at[j]], rows[p], gsem[p])

    def gwait(p):
        pltpu.make_async_copy(y0.at[pl.ds(0, ch)], rows[p], gsem[p]).wait()

    def scat(j, p):
        pltpu.async_copy(rows[p], acc.at[dstbuf.at[j]], ssem[p], add=True)

    def swait(p):
        pltpu.make_async_copy(y0.at[pl.ds(0, ch)], rows[p], ssem[p]).wait()

    ntrip = hf // 3
    for stage in range(nstage):
        pltpu.sync_copy(srcr.at[s].at[stage], srcbuf)
        pltpu.sync_copy(dstr.at[s].at[stage], dstbuf)
        gather(0, 0)
        gather(1, 1)
        if stage == 0:
            pltpu.sync_copy(zeros_h, acc.at[pl.ds(s * rpt, rpt)])
            plsc.subcore_barrier()

        def trip(kk, _):
            j = kk * 3

            gwait(0)
            scat(j, 0)

            @pl.when(kk > 0)
            def _():
                swait(2)

            gather(j + 2, 2)

            gwait(1)
            scat(j + 1, 1)

            @pl.when(kk < ntrip - 1)
            def _():
                swait(0)
                gather(j + 3, 0)

            gwait(2)
            scat(j + 2, 2)

            @pl.when(kk < ntrip - 1)
            def _():
                swait(1)
                gather(j + 4, 1)
            return 0
        lax.fori_loop(0, ntrip, trip, 0)
        swait(0)
        swait(1)
        swait(2)
    plsc.subcore_barrier()

    @pl.when(c == 0)
    def _():
        pltpu.sync_copy(acc.at[pl.ds(s * rpt, rpt)], s0.at[pl.ds(s * rpt, rpt)])

    @pl.when(c == 1)
    def _():
        pltpu.sync_copy(acc.at[pl.ds(s * rpt, rpt)], s1.at[pl.ds(s * rpt, rpt)])


def _aggregate(y0, y1, src_r, dst_r, zeros_h):
    Np, H = y0.shape
    nstage, hf, ch = src_r.shape[1], src_r.shape[2], src_r.shape[3]
    rpt = Np // NS
    fn = pl.kernel(
        functools.partial(_agg_body, nstage, rpt, ch),
        out_type=(jax.ShapeDtypeStruct((Np, H), jnp.float32),
                  jax.ShapeDtypeStruct((Np, H), jnp.float32)),
        mesh=_mesh(),
        compiler_params=_SC_PARAMS,
        scratch_types=[
            pltpu.VMEM((hf, ch), jnp.int32),
            pltpu.VMEM((hf, ch), jnp.int32),
            pltpu.VMEM((ch, H), jnp.float32),
            pltpu.VMEM((ch, H), jnp.float32),
            pltpu.VMEM((ch, H), jnp.float32),
            pltpu.SemaphoreType.DMA,
            pltpu.SemaphoreType.DMA,
            pltpu.SemaphoreType.DMA,
            pltpu.SemaphoreType.DMA,
            pltpu.SemaphoreType.DMA,
            pltpu.SemaphoreType.DMA,
            pltpu.VMEM_SHARED((Np, H), jnp.float32),
        ],
    )
    return fn(y0, y1, src_r, dst_r, zeros_h)


def _mm1_body(x_ref, w_ref, p_ref, b_ref, y0_ref, y1_ref, t_ref):
    H = y0_ref.shape[1]
    xw = jnp.dot(x_ref[...], w_ref[...], preferred_element_type=jnp.float32)
    deg = (jnp.sum(p_ref[...], axis=0) + 1.0)[:, None]
    dis = lax.rsqrt(deg)
    y = xw * dis
    y0_ref[...] = y[:, :H]
    y1_ref[...] = y[:, H:]
    t_ref[...] = xw * (dis * dis) + b_ref[...]


def _mm2_body(s0_ref, s1_ref, t1_ref, w_ref, p_ref, b_ref,
              y0_ref, y1_ref, t_ref):
    H = y0_ref.shape[1]
    deg = (jnp.sum(p_ref[...], axis=0) + 1.0)[:, None]
    dis = lax.rsqrt(deg)
    sm = jnp.concatenate([s0_ref[...], s1_ref[...]], axis=1)
    h = jnp.maximum(sm * dis + t1_ref[...], 0.0)
    xw = jnp.dot(h, w_ref[...], preferred_element_type=jnp.float32)
    y = xw * dis
    y0_ref[...] = y[:, :H]
    y1_ref[...] = y[:, H:]
    t_ref[...] = xw * (dis * dis) + b_ref[...]


def _pool_body(nb, s0_ref, s1_ref, t2_ref, p_ref, b_ref, out_ref,
               acc_ref, cnt_ref):
    i = pl.program_id(0)
    blk = s0_ref.shape[0]
    deg = (jnp.sum(p_ref[...], axis=0) + 1.0)[:, None]
    dis = lax.rsqrt(deg)
    h = jnp.concatenate([s0_ref[...], s1_ref[...]], axis=1) * dis + t2_ref[...]
    bvec = b_ref[0, 0, :]
    gids = lax.broadcasted_iota(jnp.int32, (1, G), 1)
    A = (bvec[:, None] == gids).astype(jnp.float32)
    onec = jnp.ones((blk, 1), jnp.float32)

    @pl.when(i == 0)
    def _():
        acc_ref[...] = jnp.zeros_like(acc_ref)
        cnt_ref[...] = jnp.zeros_like(cnt_ref)

    acc_ref[...] += lax.dot_general(A, h, (((0,), (0,)), ((), ())),
                                    preferred_element_type=jnp.float32)
    cnt_ref[...] += lax.dot_general(A, onec, (((0,), (0,)), ((), ())),
                                    preferred_element_type=jnp.float32)

    @pl.when(i == nb - 1)
    def _():
        out_ref[...] = acc_ref[...] / jnp.maximum(cnt_ref[...], 1.0)


def _layer_mm(first, Np, *args):
    D = args[1].shape[0] if first else args[3].shape[0]
    H = D // 2
    nb = Np // BLK
    if first:
        body = _mm1_body
        specs = [
            pl.BlockSpec((BLK, D), lambda i: (i, 0)),
            pl.BlockSpec((D, D), lambda i: (0, 0)),
            pl.BlockSpec((NW, BLK), lambda i: (0, i)),
            pl.BlockSpec((1, D), lambda i: (0, 0)),
        ]
    else:
        body = _mm2_body
        specs = [
            pl.BlockSpec((BLK, H), lambda i: (i, 0)),
            pl.BlockSpec((BLK, H), lambda i: (i, 0)),
            pl.BlockSpec((BLK, D), lambda i: (i, 0)),
            pl.BlockSpec((D, D), lambda i: (0, 0)),
            pl.BlockSpec((NW, BLK), lambda i: (0, i)),
            pl.BlockSpec((1, D), lambda i: (0, 0)),
        ]
    return pl.pallas_call(
        body,
        grid=(nb,),
        in_specs=specs,
        out_specs=[
            pl.BlockSpec((BLK, H), lambda i: (i, 0)),
            pl.BlockSpec((BLK, H), lambda i: (i, 0)),
            pl.BlockSpec((BLK, D), lambda i: (i, 0)),
        ],
        out_shape=[
            jax.ShapeDtypeStruct((Np, H), jnp.float32),
            jax.ShapeDtypeStruct((Np, H), jnp.float32),
            jax.ShapeDtypeStruct((Np, D), jnp.float32),
        ],
    )(*args)


def _pool(s0, s1, t2, parts, batch3):
    Np, H = s0.shape
    D = 2 * H
    nb = Np // BLK
    return pl.pallas_call(
        functools.partial(_pool_body, nb),
        grid=(nb,),
        in_specs=[
            pl.BlockSpec((BLK, H), lambda i: (i, 0)),
            pl.BlockSpec((BLK, H), lambda i: (i, 0)),
            pl.BlockSpec((BLK, D), lambda i: (i, 0)),
            pl.BlockSpec((NW, BLK), lambda i: (0, i)),
            pl.BlockSpec((1, 1, BLK), lambda i: (i, 0, 0)),
        ],
        out_specs=pl.BlockSpec((G, D), lambda i: (0, 0)),
        out_shape=jax.ShapeDtypeStruct((G, D), jnp.float32),
        scratch_shapes=[
            pltpu.VMEM((G, D), jnp.float32),
            pltpu.VMEM((G, 1), jnp.float32),
        ],
    )(s0, s1, t2, parts, batch3)


def kernel(x, edge_index, batch, W1, b1, W2, b2):
    N, D = x.shape
    E = edge_index.shape[1]
    H = D // 2
    Np = (N + BLK - 1) // BLK * BLK
    if Np == N:
        Np += BLK
    CH = 96
    ept = -(-E // (NS * 36 * CH)) * 36 * CH
    Ep = ept * NS
    nchk = ept // CH
    assert nchk % 9 == 0 and Np % (NS * 8) == 0 and Ep % NW == 0

    npad = Ep - E
    pad_src = jnp.asarray(np.arange(npad, dtype=np.int32) % N)
    pad_dst = jnp.asarray(N + np.arange(npad, dtype=np.int32) % (Np - N))
    src = jnp.concatenate([edge_index[0], pad_src])
    dst = jnp.concatenate([edge_index[1], pad_dst])
    nstage = 3
    src_r = src.reshape(NS, nstage, nchk // nstage, CH)
    dst_r = dst.reshape(NS, nstage, nchk // nstage, CH)
    zeros_h = jnp.zeros((Np // NS, H), jnp.float32)
    b1r = b1.reshape(1, D)
    b2r = b2.reshape(1, D)
    batch3 = jnp.pad(batch, (0, Np - N), constant_values=G).reshape(
        Np // BLK, 1, BLK)

    parts = _degree_partials(dst, Np)
    y0, y1, t1 = _layer_mm(True, Np, x, W1, parts, b1r)
    s0, s1 = _aggregate(y0, y1, src_r, dst_r, zeros_h)
    y0b, y1b, t2 = _layer_mm(False, Np, s0, s1, t1, W2, parts, b2r)
    s0b, s1b = _aggregate(y0b, y1b, src_r, dst_r, zeros_h)
    return _pool(s0b, s1b, t2, parts, batch3)


# speedup vs baseline: 1.0203x; 1.0203x over previous
import functools

import numpy as np

import jax
import jax.numpy as jnp
from jax import lax
from jax.experimental import pallas as pl
from jax.experimental.pallas import tpu as pltpu
from jax.experimental.pallas import tpu_sc as plsc

NC = 2
NS = 16
NW = NC * NS
L = 16
G = 64
BLK = 1024


@functools.lru_cache(maxsize=1)
def _mesh():
    return plsc.VectorSubcoreMesh(core_axis_name="c", subcore_axis_name="s",
                                  num_cores=NC, num_subcores=NS)

_SC_PARAMS = pltpu.CompilerParams(needs_layout_passes=False)


def _deg_body(E, Np, dst_hbm, out_hbm, dstbuf, bins):
    epw = E // NW
    c = lax.axis_index("c")
    s = lax.axis_index("s")
    w = s * NC + c
    pltpu.sync_copy(dst_hbm.at[pl.ds(w * epw, epw)], dstbuf.at[pl.ds(0, epw)])

    def zero(i, _):
        bins[pl.ds(i * L, L)] = jnp.zeros((L,), jnp.float32)
        return 0
    lax.fori_loop(0, Np // L, zero, 0)

    ones = jnp.ones((L,), jnp.float32)

    def upd(i, _):
        idx = dstbuf[pl.ds(i * L, L)]
        plsc.addupdate_scatter(bins, [idx], ones)
        return 0
    nfull = epw // L
    lax.fori_loop(0, nfull, upd, 0)
    rem = epw - nfull * L
    if rem:
        idx = dstbuf[pl.ds(nfull * L, L)]
        mask = lax.iota(jnp.int32, L) < rem
        plsc.addupdate_scatter(bins, [idx], ones, mask=mask)
    pltpu.sync_copy(bins, out_hbm.at[w])


def _degree_partials(dst, Np):
    E = dst.shape[0]
    epw = E // NW
    pad = (-epw) % L
    fn = pl.kernel(
        functools.partial(_deg_body, E, Np),
        out_type=jax.ShapeDtypeStruct((NW, Np), jnp.float32),
        mesh=_mesh(),
        compiler_params=_SC_PARAMS,
        scratch_types=[
            pltpu.VMEM((epw + pad,), jnp.int32),
            pltpu.VMEM((Np,), jnp.float32),
        ],
    )
    return fn(dst)


def _agg_body(nstage, rpt, ch, y0, y1, srcr, dstr, zeros_h, s0, s1,
              srcbuf, dstbuf, rows0, rows1, rows2,
              g0, g1, g2, t0, t1, t2, acc):
    c = lax.axis_index("c")
    s = lax.axis_index("s")
    hf = srcbuf.shape[0]
    rows = (rows0, rows1, rows2)
    gsem = (g0, g1, g2)
    ssem = (t0, t1, t2)

    def gather(j, p):
        @pl.when(c == 0)
        def _():
            pltpu.async_copy(y0.at[srcbuf.at[j]], rows[p], gsem[p])

        @pl.when(c == 1)
        def _():
            pltpu.async_copy(y1.at[srcbuf.at[j]], rows[p], gsem[p])

    def gwait(p):
        pltpu.make_async_copy(y0.at[pl.ds(0, ch)], rows[p], gsem[p]).wait()

    def scat(j, p):
        pltpu.async_copy(rows[p], acc.at[dstbuf.at[j]], ssem[p], add=True)

    def swait(p):
        pltpu.make_async_copy(y0.at[pl.ds(0, ch)], rows[p], ssem[p]).wait()

    ntrip = hf // 3
    for stage in range(nstage):
        pltpu.sync_copy(srcr.at[s].at[stage], srcbuf)
        pltpu.sync_copy(dstr.at[s].at[stage], dstbuf)
        gather(0, 0)
        gather(1, 1)
        if stage == 0:
            pltpu.sync_copy(zeros_h, acc.at[pl.ds(s * rpt, rpt)])
            plsc.subcore_barrier()

        def trip(kk, _):
            j = kk * 3

            gwait(0)
            scat(j, 0)

            @pl.when(kk > 0)
            def _():
                swait(2)

            gather(j + 2, 2)

            gwait(1)
            scat(j + 1, 1)

            @pl.when(kk < ntrip - 1)
            def _():
                swait(0)
                gather(j + 3, 0)

            gwait(2)
            scat(j + 2, 2)

            @pl.when(kk < ntrip - 1)
            def _():
                swait(1)
                gather(j + 4, 1)
            return 0
        lax.fori_loop(0, ntrip, trip, 0)
        swait(0)
        swait(1)
        swait(2)
    plsc.subcore_barrier()

    @pl.when(c == 0)
    def _():
        pltpu.sync_copy(acc.at[pl.ds(s * rpt, rpt)], s0.at[pl.ds(s * rpt, rpt)])

    @pl.when(c == 1)
    def _():
        pltpu.sync_copy(acc.at[pl.ds(s * rpt, rpt)], s1.at[pl.ds(s * rpt, rpt)])


def _aggregate(y0, y1, src_r, dst_r, zeros_h):
    Np, H = y0.shape
    nstage, hf, ch = src_r.shape[1], src_r.shape[2], src_r.shape[3]
    rpt = Np // NS
    fn = pl.kernel(
        functools.partial(_agg_body, nstage, rpt, ch),
        out_type=(jax.ShapeDtypeStruct((Np, H), jnp.float32),
                  jax.ShapeDtypeStruct((Np, H), jnp.float32)),
        mesh=_mesh(),
        compiler_params=_SC_PARAMS,
        scratch_types=[
            pltpu.VMEM((hf, ch), jnp.int32),
            pltpu.VMEM((hf, ch), jnp.int32),
            pltpu.VMEM((ch, H), jnp.float32),
            pltpu.VMEM((ch, H), jnp.float32),
            pltpu.VMEM((ch, H), jnp.float32),
            pltpu.SemaphoreType.DMA,
            pltpu.SemaphoreType.DMA,
            pltpu.SemaphoreType.DMA,
            pltpu.SemaphoreType.DMA,
            pltpu.SemaphoreType.DMA,
            pltpu.SemaphoreType.DMA,
            pltpu.VMEM_SHARED((Np, H), jnp.float32),
        ],
    )
    return fn(y0, y1, src_r, dst_r, zeros_h)


def _mm1_body(x_ref, w_ref, p_ref, b_ref, y0_ref, y1_ref, t_ref):
    H = y0_ref.shape[1]
    xw = jnp.dot(x_ref[...], w_ref[...], preferred_element_type=jnp.float32)
    deg = (jnp.sum(p_ref[...], axis=0) + 1.0)[:, None]
    dis = lax.rsqrt(deg)
    y = xw * dis
    y0_ref[...] = y[:, :H]
    y1_ref[...] = y[:, H:]
    t_ref[...] = xw * (dis * dis) + b_ref[...]


def _mm2_body(s0_ref, s1_ref, t1_ref, w_ref, p_ref, b_ref,
              y0_ref, y1_ref, t_ref):
    H = y0_ref.shape[1]
    deg = (jnp.sum(p_ref[...], axis=0) + 1.0)[:, None]
    dis = lax.rsqrt(deg)
    sm = jnp.concatenate([s0_ref[...], s1_ref[...]], axis=1)
    h = jnp.maximum(sm * dis + t1_ref[...], 0.0)
    xw = jnp.dot(h, w_ref[...], preferred_element_type=jnp.float32)
    y = xw * dis
    y0_ref[...] = y[:, :H]
    y1_ref[...] = y[:, H:]
    t_ref[...] = xw * (dis * dis) + b_ref[...]


def _pool_body(nb, s0_ref, s1_ref, t2_ref, p_ref, b_ref, out_ref,
               acc_ref, cnt_ref):
    i = pl.program_id(0)
    blk = s0_ref.shape[0]
    deg = (jnp.sum(p_ref[...], axis=0) + 1.0)[:, None]
    dis = lax.rsqrt(deg)
    h = jnp.concatenate([s0_ref[...], s1_ref[...]], axis=1) * dis + t2_ref[...]
    bvec = b_ref[0, 0, :]
    gids = lax.broadcasted_iota(jnp.int32, (1, G), 1)
    A = (bvec[:, None] == gids).astype(jnp.float32)
    onec = jnp.ones((blk, 1), jnp.float32)

    @pl.when(i == 0)
    def _():
        acc_ref[...] = jnp.zeros_like(acc_ref)
        cnt_ref[...] = jnp.zeros_like(cnt_ref)

    acc_ref[...] += lax.dot_general(A, h, (((0,), (0,)), ((), ())),
                                    preferred_element_type=jnp.float32)
    cnt_ref[...] += lax.dot_general(A, onec, (((0,), (0,)), ((), ())),
                                    preferred_element_type=jnp.float32)

    @pl.when(i == nb - 1)
    def _():
        out_ref[...] = acc_ref[...] / jnp.maximum(cnt_ref[...], 1.0)


def _layer_mm(first, Np, *args):
    D = args[1].shape[0] if first else args[3].shape[0]
    H = D // 2
    nb = Np // BLK
    if first:
        body = _mm1_body
        specs = [
            pl.BlockSpec((BLK, D), lambda i: (i, 0)),
            pl.BlockSpec((D, D), lambda i: (0, 0)),
            pl.BlockSpec((NW, BLK), lambda i: (0, i)),
            pl.BlockSpec((1, D), lambda i: (0, 0)),
        ]
    else:
        body = _mm2_body
        specs = [
            pl.BlockSpec((BLK, H), lambda i: (i, 0)),
            pl.BlockSpec((BLK, H), lambda i: (i, 0)),
            pl.BlockSpec((BLK, D), lambda i: (i, 0)),
            pl.BlockSpec((D, D), lambda i: (0, 0)),
            pl.BlockSpec((NW, BLK), lambda i: (0, i)),
            pl.BlockSpec((1, D), lambda i: (0, 0)),
        ]
    return pl.pallas_call(
        body,
        grid=(nb,),
        in_specs=specs,
        out_specs=[
            pl.BlockSpec((BLK, H), lambda i: (i, 0)),
            pl.BlockSpec((BLK, H), lambda i: (i, 0)),
            pl.BlockSpec((BLK, D), lambda i: (i, 0)),
        ],
        out_shape=[
            jax.ShapeDtypeStruct((Np, H), jnp.float32),
            jax.ShapeDtypeStruct((Np, H), jnp.float32),
            jax.ShapeDtypeStruct((Np, D), jnp.float32),
        ],
    )(*args)


def _pool(s0, s1, t2, parts, batch3):
    Np, H = s0.shape
    D = 2 * H
    nb = Np // BLK
    return pl.pallas_call(
        functools.partial(_pool_body, nb),
        grid=(nb,),
        in_specs=[
            pl.BlockSpec((BLK, H), lambda i: (i, 0)),
            pl.BlockSpec((BLK, H), lambda i: (i, 0)),
            pl.BlockSpec((BLK, D), lambda i: (i, 0)),
            pl.BlockSpec((NW, BLK), lambda i: (0, i)),
            pl.BlockSpec((1, 1, BLK), lambda i: (i, 0, 0)),
        ],
        out_specs=pl.BlockSpec((G, D), lambda i: (0, 0)),
        out_shape=jax.ShapeDtypeStruct((G, D), jnp.float32),
        scratch_shapes=[
            pltpu.VMEM((G, D), jnp.float32),
            pltpu.VMEM((G, 1), jnp.float32),
        ],
    )(s0, s1, t2, parts, batch3)


def kernel(x, edge_index, batch, W1, b1, W2, b2):
    N, D = x.shape
    E = edge_index.shape[1]
    H = D // 2
    Np = (N + BLK - 1) // BLK * BLK
    if Np == N:
        Np += BLK
    CH = 80
    ept = -(-E // (NS * 9 * CH)) * 9 * CH
    Ep = ept * NS
    nchk = ept // CH
    assert nchk % 9 == 0 and Np % (NS * 8) == 0 and Ep % NW == 0

    npad = Ep - E
    pad_src = jnp.asarray(np.arange(npad, dtype=np.int32) % N)
    pad_dst = jnp.asarray(N + np.arange(npad, dtype=np.int32) % (Np - N))
    src = jnp.concatenate([edge_index[0], pad_src])
    dst = jnp.concatenate([edge_index[1], pad_dst])
    nstage = 3
    src_r = src.reshape(NS, nstage, nchk // nstage, CH)
    dst_r = dst.reshape(NS, nstage, nchk // nstage, CH)
    zeros_h = jnp.zeros((Np // NS, H), jnp.float32)
    b1r = b1.reshape(1, D)
    b2r = b2.reshape(1, D)
    batch3 = jnp.pad(batch, (0, Np - N), constant_values=G).reshape(
        Np // BLK, 1, BLK)

    parts = _degree_partials(dst, Np)
    y0, y1, t1 = _layer_mm(True, Np, x, W1, parts, b1r)
    s0, s1 = _aggregate(y0, y1, src_r, dst_r, zeros_h)
    y0b, y1b, t2 = _layer_mm(False, Np, s0, s1, t1, W2, parts, b2r)
    s0b, s1b = _aggregate(y0b, y1b, src_r, dst_r, zeros_h)
    return _pool(s0b, s1b, t2, parts, batch3)
